# baseline (speedup 1.0000x reference)
import jax
import jax.numpy as jnp
from jax.experimental import pallas as pl
from jax.experimental.pallas import tpu as pltpu

N = 10000
D = 128
BM = 400
G = N // BM


def _dot_t(x, w):
    return jax.lax.dot_general(x, w, (((1,), (1,)), ((), ())),
                               preferred_element_type=jnp.float32)


def _fused_gcn_kernel(a_hbm, ax_ref, wr_ref, wrb_ref, w_ref, wb_ref,
                      out_ref, h2_ref, a0_ref, a1_ref, sem):
    i = pl.program_id(0)
    even = (i % 2) == 0

    def a_copy(blk, buf_ref, slot):
        return pltpu.make_async_copy(
            a_hbm.at[pl.ds(blk * BM, BM), :], buf_ref, sem.at[slot])

    @pl.when(i == 0)
    def _prologue():
        a_copy(0, a0_ref, 0).start()
        a_copy(1, a1_ref, 1).start()
        h = _dot_t(ax_ref[...], wr_ref[...]) + wrb_ref[...][None, :]
        h2_ref[...] = _dot_t(jnp.maximum(h, 0.0), w_ref[...])

    interior = jnp.logical_and(i > 0, i < G - 1)

    @pl.when(jnp.logical_and(interior, even))
    def _prefetch_to_odd():
        a_copy(i + 1, a1_ref, 1).start()

    @pl.when(jnp.logical_and(interior, jnp.logical_not(even)))
    def _prefetch_to_even():
        a_copy(i + 1, a0_ref, 0).start()

    def _consume(buf_ref, slot):
        a_copy(i, buf_ref, slot).wait()
        temp = jnp.dot(buf_ref[...], h2_ref[...],
                       preferred_element_type=jnp.float32)
        out_ref[0] = jnp.maximum(temp + wb_ref[...][None, :], 0.0)

    @pl.when(even)
    def _consume_even():
        _consume(a0_ref, 0)

    @pl.when(jnp.logical_not(even))
    def _consume_odd():
        _consume(a1_ref, 1)


@jax.jit
def _run(A, AX, Wr, Wr_b, W, W_b):
    out = pl.pallas_call(
        _fused_gcn_kernel,
        grid=(G,),
        in_specs=[
            pl.BlockSpec(memory_space=pltpu.MemorySpace.HBM),
            pl.BlockSpec((N, D), lambda i: (0, 0)),
            pl.BlockSpec((D, D), lambda i: (0, 0)),
            pl.BlockSpec((D,), lambda i: (0,)),
            pl.BlockSpec((D, D), lambda i: (0, 0)),
            pl.BlockSpec((D,), lambda i: (0,)),
        ],
        out_specs=pl.BlockSpec((1, BM, D), lambda i: (0, i, 0)),
        out_shape=jax.ShapeDtypeStruct((1, N, D), jnp.float32),
        scratch_shapes=[
            pltpu.VMEM((N, D), jnp.float32),
            pltpu.VMEM((BM, N), jnp.float32),
            pltpu.VMEM((BM, N), jnp.float32),
            pltpu.SemaphoreType.DMA((2,)),
        ],
        compiler_params=pltpu.CompilerParams(
            dimension_semantics=("arbitrary",),
        ),
    )(A, AX, Wr, Wr_b, W, W_b)
    return out


def kernel(A, AX, Wr_w, Wr_b, W_w, W_b):
    return _run(A, AX, Wr_w, Wr_b, W_w, W_b)


# speedup vs baseline: 1.0058x; 1.0058x over previous
import jax
import jax.numpy as jnp
from jax.experimental import pallas as pl
from jax.experimental.pallas import tpu as pltpu

N = 10000
D = 128
BM = 400


def _dot_t(x, w):
    return jax.lax.dot_general(x, w, (((1,), (1,)), ((), ())),
                               preferred_element_type=jnp.float32)


def _fused_gcn_kernel(a_ref, ax_ref, wr_ref, wrb_ref, w_ref, wb_ref,
                      out_ref, h2_ref):
    @pl.when(pl.program_id(0) == 0)
    def _compute_h2():
        h = _dot_t(ax_ref[...], wr_ref[...]) + wrb_ref[...][None, :]
        h2_ref[...] = _dot_t(jnp.maximum(h, 0.0), w_ref[...])

    temp = jnp.dot(a_ref[...], h2_ref[...], preferred_element_type=jnp.float32)
    out_ref[0] = jnp.maximum(temp + wb_ref[...][None, :], 0.0)


@jax.jit
def _run(A, AX, Wr, Wr_b, W, W_b):
    out = pl.pallas_call(
        _fused_gcn_kernel,
        grid=(N // BM,),
        in_specs=[
            pl.BlockSpec((BM, N), lambda i: (i, 0)),
            pl.BlockSpec((N, D), lambda i: (0, 0)),
            pl.BlockSpec((D, D), lambda i: (0, 0)),
            pl.BlockSpec((D,), lambda i: (0,)),
            pl.BlockSpec((D, D), lambda i: (0, 0)),
            pl.BlockSpec((D,), lambda i: (0,)),
        ],
        out_specs=pl.BlockSpec((1, BM, D), lambda i: (0, i, 0)),
        out_shape=jax.ShapeDtypeStruct((1, N, D), jnp.float32),
        scratch_shapes=[pltpu.VMEM((N, D), jnp.float32)],
        compiler_params=pltpu.CompilerParams(
            dimension_semantics=("arbitrary",),
        ),
    )(A, AX, Wr, Wr_b, W, W_b)
    return out


def kernel(A, AX, Wr_w, Wr_b, W_w, W_b):
    return _run(A, AX, Wr_w, Wr_b, W_w, W_b)
